# Initial kernel scaffold; baseline (speedup 1.0000x reference)
import jax
import jax.numpy as jnp
from jax.experimental import pallas as pl


def kernel(inputs, embeddings):
    raise NotImplementedError("write your pallas kernel here")


# speedup vs baseline: 224.0072x; 224.0072x over previous
import jax
import jax.numpy as jnp
from jax.experimental import pallas as pl

_BB = 128


def _bag_matmul_kernel(x_ref, e_ref, o_ref):
    o_ref[...] = jnp.dot(x_ref[...], e_ref[...],
                         preferred_element_type=jnp.float32)


def kernel(inputs, embeddings):
    B, V = inputs.shape
    _, D = embeddings.shape
    return pl.pallas_call(
        _bag_matmul_kernel,
        grid=(B // _BB,),
        in_specs=[
            pl.BlockSpec((_BB, V), lambda i: (i, 0)),
            pl.BlockSpec((V, D), lambda i: (0, 0)),
        ],
        out_specs=pl.BlockSpec((_BB, D), lambda i: (i, 0)),
        out_shape=jax.ShapeDtypeStruct((B, D), jnp.float32),
    )(inputs, embeddings)


# speedup vs baseline: 279.2001x; 1.2464x over previous
import jax
import jax.numpy as jnp
from jax.experimental import pallas as pl

_BB = 512


def _bag_matmul_kernel(x_ref, e_ref, o_ref):
    o_ref[...] = jnp.dot(x_ref[...], e_ref[...],
                         preferred_element_type=jnp.float32)


def kernel(inputs, embeddings):
    B, V = inputs.shape
    _, D = embeddings.shape
    return pl.pallas_call(
        _bag_matmul_kernel,
        grid=(B // _BB,),
        in_specs=[
            pl.BlockSpec((_BB, V), lambda i: (i, 0)),
            pl.BlockSpec((V, D), lambda i: (0, 0)),
        ],
        out_specs=pl.BlockSpec((_BB, D), lambda i: (i, 0)),
        out_shape=jax.ShapeDtypeStruct((B, D), jnp.float32),
    )(inputs, embeddings)
